# speedup vs baseline: 11.7435x; 11.7435x over previous
import functools

import jax
import jax.numpy as jnp
from jax import lax
from jax.experimental import pallas as pl
from jax.experimental.pallas import tpu as pltpu
from jax.experimental.pallas import tpu_sc as plsc

N_NODES = 10000
N_EDGES = 320000
D = 128

NC = 2
NS = 16
NW = NC * NS

NPAD = 10240
RPT = NPAD // NS
CHUNK = 128
EPT_CHUNKS = 80
EPT = CHUNK * EPT_CHUNKS
EPAD = EPT * NW
BR = 1024


def _sc_degree(dst_idx, zeros_col, ones_chunk):
    mesh = plsc.VectorSubcoreMesh(core_axis_name="c", subcore_axis_name="s")

    @functools.partial(
        pl.kernel,
        out_type=jax.ShapeDtypeStruct((NC, NPAD), jnp.float32),
        mesh=mesh,
        scratch_types=[
            pltpu.VMEM((CHUNK,), jnp.int32),
            pltpu.VMEM((CHUNK,), jnp.float32),
            pltpu.VMEM_SHARED((NPAD,), jnp.float32),
        ],
    )
    def deg_kernel(dst_hbm, z_hbm, one_hbm, out_hbm, didx_v, ones_v, deg_sh):
        c = lax.axis_index("c")
        s = lax.axis_index("s")
        wid = c * NS + s
        pltpu.sync_copy(z_hbm, deg_sh.at[pl.ds(s * RPT, RPT)])
        pltpu.sync_copy(one_hbm, ones_v)
        plsc.subcore_barrier()

        @pl.loop(0, EPT_CHUNKS)
        def _(i):
            base = wid * EPT + i * CHUNK
            pltpu.sync_copy(dst_hbm.at[pl.ds(base, CHUNK)], didx_v)
            pltpu.sync_copy(ones_v, deg_sh.at[didx_v], add=True)

        plsc.subcore_barrier()
        pltpu.sync_copy(deg_sh.at[pl.ds(s * RPT, RPT)],
                        out_hbm.at[c, pl.ds(s * RPT, RPT)])

    return deg_kernel(dst_idx, zeros_col, ones_chunk)


def _sc_scatter(y, src_idx, dst_idx, zeros_rows):
    mesh = plsc.VectorSubcoreMesh(core_axis_name="c", subcore_axis_name="s")

    @functools.partial(
        pl.kernel,
        out_type=jax.ShapeDtypeStruct((NC, NPAD, D), jnp.float32),
        mesh=mesh,
        scratch_types=[
            pltpu.VMEM((CHUNK,), jnp.int32),
            pltpu.VMEM((CHUNK,), jnp.int32),
            pltpu.VMEM((CHUNK, D), jnp.float32),
            pltpu.VMEM_SHARED((NPAD, D), jnp.float32),
        ],
    )
    def scatter_kernel(y_hbm, src_hbm, dst_hbm, z_hbm, out_hbm,
                       sidx_v, didx_v, rows_v, agg_sh):
        c = lax.axis_index("c")
        s = lax.axis_index("s")
        wid = c * NS + s
        pltpu.sync_copy(z_hbm, agg_sh.at[pl.ds(s * RPT, RPT)])
        plsc.subcore_barrier()

        @pl.loop(0, EPT_CHUNKS)
        def _(i):
            base = wid * EPT + i * CHUNK
            pltpu.sync_copy(src_hbm.at[pl.ds(base, CHUNK)], sidx_v)
            pltpu.sync_copy(dst_hbm.at[pl.ds(base, CHUNK)], didx_v)
            pltpu.sync_copy(y_hbm.at[sidx_v], rows_v)
            pltpu.sync_copy(rows_v, agg_sh.at[didx_v], add=True)

        plsc.subcore_barrier()
        pltpu.sync_copy(agg_sh.at[pl.ds(s * RPT, RPT)],
                        out_hbm.at[c, pl.ds(s * RPT, RPT)])

    return scatter_kernel(y, src_idx, dst_idx, zeros_rows)


def _matmul_body(x_ref, w_ref, deg_ref, y_ref):
    dsum = deg_ref[:, 0:1] + deg_ref[:, 1:2] + 1.0
    xw = jnp.dot(x_ref[...], w_ref[...], preferred_element_type=jnp.float32)
    y_ref[...] = xw * lax.rsqrt(dsum)


def _tc_scale_matmul(x_pad, W, deg_t):
    return pl.pallas_call(
        _matmul_body,
        grid=(NPAD // BR,),
        in_specs=[
            pl.BlockSpec((BR, D), lambda i: (i, 0)),
            pl.BlockSpec((D, D), lambda i: (0, 0)),
            pl.BlockSpec((BR, NC), lambda i: (i, 0)),
        ],
        out_specs=pl.BlockSpec((BR, D), lambda i: (i, 0)),
        out_shape=jax.ShapeDtypeStruct((NPAD, D), jnp.float32),
    )(x_pad, W, deg_t)


def _final_body(agg_ref, y_ref, deg_ref, b_ref, g_ref, bt_ref, o_ref):
    dsum = deg_ref[:, 0:1] + deg_ref[:, 1:2] + 1.0
    t = (agg_ref[0] + agg_ref[1] + y_ref[...]) * lax.rsqrt(dsum) + b_ref[...]
    mean = jnp.mean(t, axis=-1, keepdims=True)
    ctr = t - mean
    var = jnp.mean(ctr * ctr, axis=-1, keepdims=True)
    o_ref[...] = ctr * lax.rsqrt(var + 1e-5) * g_ref[...] + bt_ref[...]


def _tc_final(agg_parts, y, deg_t, b2, g2, bt2):
    return pl.pallas_call(
        _final_body,
        grid=(NPAD // BR,),
        in_specs=[
            pl.BlockSpec((NC, BR, D), lambda i: (0, i, 0)),
            pl.BlockSpec((BR, D), lambda i: (i, 0)),
            pl.BlockSpec((BR, NC), lambda i: (i, 0)),
            pl.BlockSpec((1, D), lambda i: (0, 0)),
            pl.BlockSpec((1, D), lambda i: (0, 0)),
            pl.BlockSpec((1, D), lambda i: (0, 0)),
        ],
        out_specs=pl.BlockSpec((BR, D), lambda i: (i, 0)),
        out_shape=jax.ShapeDtypeStruct((NPAD, D), jnp.float32),
    )(agg_parts, y, deg_t, b2, g2, bt2)


def kernel(x, edge_index, W, b, gamma, beta):
    src = edge_index[0].astype(jnp.int32)
    dst = edge_index[1].astype(jnp.int32)
    pad_e = EPAD - N_EDGES
    src_pad = jnp.concatenate([src, jnp.zeros((pad_e,), jnp.int32)])
    dst_pad = jnp.concatenate([dst, jnp.full((pad_e,), NPAD - 1, jnp.int32)])

    zeros_col = jnp.zeros((RPT,), jnp.float32)
    ones_chunk = jnp.ones((CHUNK,), jnp.float32)
    zeros_rows = jnp.zeros((RPT, D), jnp.float32)

    deg_parts = _sc_degree(dst_pad, zeros_col, ones_chunk)
    deg_t = deg_parts.T

    x_pad = jnp.concatenate([x, jnp.zeros((NPAD - N_NODES, D), x.dtype)])
    y = _tc_scale_matmul(x_pad, W, deg_t)
    agg_parts = _sc_scatter(y, src_pad, dst_pad, zeros_rows)
    out = _tc_final(agg_parts, y, deg_t,
                    b.reshape(1, D), gamma.reshape(1, D), beta.reshape(1, D))
    return out[:N_NODES]


# speedup vs baseline: 34.7785x; 2.9615x over previous
import functools

import jax
import jax.numpy as jnp
from jax import lax
from jax.experimental import pallas as pl
from jax.experimental.pallas import tpu as pltpu
from jax.experimental.pallas import tpu_sc as plsc

N_NODES = 10000
N_EDGES = 320000
D = 128

NC = 2
NS = 16
NW = NC * NS

NPAD = 10240
RPT = NPAD // NS
CHUNK = 128
EPT_CHUNKS = 80
EPT = CHUNK * EPT_CHUNKS
EPAD = EPT * NW
BR = 1024


def _sc_degree(dst_idx3, zeros_col, ones_chunk):
    mesh = plsc.VectorSubcoreMesh(core_axis_name="c", subcore_axis_name="s")

    @functools.partial(
        pl.kernel,
        out_type=jax.ShapeDtypeStruct((NC, NPAD), jnp.float32),
        mesh=mesh,
        scratch_types=[
            pltpu.VMEM((EPT_CHUNKS, CHUNK), jnp.int32),
            pltpu.VMEM((CHUNK,), jnp.float32),
            pltpu.SemaphoreType.DMA,
            pltpu.VMEM_SHARED((NPAD,), jnp.float32),
        ],
    )
    def deg_kernel(dst_hbm, z_hbm, one_hbm, out_hbm, didx_all, ones_v, sem, deg_sh):
        c = lax.axis_index("c")
        s = lax.axis_index("s")
        wid = c * NS + s
        pltpu.sync_copy(dst_hbm.at[wid], didx_all)
        pltpu.sync_copy(z_hbm, deg_sh.at[pl.ds(s * RPT, RPT)])
        pltpu.sync_copy(one_hbm, ones_v)
        plsc.subcore_barrier()

        @pl.loop(0, EPT_CHUNKS)
        def _(i):
            pltpu.async_copy(ones_v, deg_sh.at[didx_all.at[i]], sem, add=True)

        @pl.loop(0, EPT_CHUNKS)
        def _(i):
            pltpu.make_async_copy(ones_v, deg_sh.at[didx_all.at[i]], sem).wait()

        plsc.subcore_barrier()
        pltpu.sync_copy(deg_sh.at[pl.ds(s * RPT, RPT)],
                        out_hbm.at[c, pl.ds(s * RPT, RPT)])

    return deg_kernel(dst_idx3, zeros_col, ones_chunk)


def _sc_scatter(y, src_idx3, dst_idx3, zeros_rows):
    mesh = plsc.VectorSubcoreMesh(core_axis_name="c", subcore_axis_name="s")

    @functools.partial(
        pl.kernel,
        out_type=jax.ShapeDtypeStruct((NC, NPAD, D), jnp.float32),
        mesh=mesh,
        scratch_types=[
            pltpu.VMEM((EPT,), jnp.int32),
            pltpu.VMEM((CHUNK,), jnp.int32),
            pltpu.VMEM((CHUNK,), jnp.int32),
            pltpu.VMEM((CHUNK, D), jnp.float32),
            pltpu.VMEM((CHUNK, D), jnp.float32),
            pltpu.SemaphoreType.DMA,
            pltpu.SemaphoreType.DMA,
            pltpu.SemaphoreType.DMA,
            pltpu.SemaphoreType.DMA,
            pltpu.SemaphoreType.DMA,
            pltpu.SemaphoreType.DMA,
            pltpu.VMEM_SHARED((NPAD, D), jnp.float32),
        ],
    )
    def scatter_kernel(y_hbm, src_hbm, dst_hbm, z_hbm, out_hbm,
                       sidx_all, didx0, didx1, rows0, rows1,
                       sg0, sg1, ss0, ss1, sd0, sd1, agg_sh):
        c = lax.axis_index("c")
        s = lax.axis_index("s")
        wid = c * NS + s
        pltpu.sync_copy(src_hbm.at[wid], sidx_all)
        pltpu.sync_copy(z_hbm, agg_sh.at[pl.ds(s * RPT, RPT)])
        plsc.subcore_barrier()

        def g_start(k, buf, sem):
            pltpu.async_copy(y_hbm.at[sidx_all.at[pl.ds(k * CHUNK, CHUNK)]],
                             buf, sem)

        def g_wait(k, buf, sem):
            pltpu.make_async_copy(y_hbm.at[sidx_all.at[pl.ds(k * CHUNK, CHUNK)]],
                                  buf, sem).wait()

        def d_start(k, dbuf, sem):
            pltpu.async_copy(dst_hbm.at[wid, k], dbuf, sem)

        def d_wait(k, dbuf, sem):
            pltpu.make_async_copy(dst_hbm.at[wid, k], dbuf, sem).wait()

        def s_start(buf, dbuf, sem):
            pltpu.async_copy(buf, agg_sh.at[dbuf], sem, add=True)

        def s_wait(buf, dbuf, sem):
            pltpu.make_async_copy(buf, agg_sh.at[dbuf], sem).wait()

        g_start(0, rows0, sg0)
        d_start(0, didx0, sd0)
        g_start(1, rows1, sg1)
        d_start(1, didx1, sd1)

        @pl.loop(0, EPT_CHUNKS // 2 - 1)
        def _(i):
            k = i * 2
            g_wait(k, rows0, sg0)
            d_wait(k, didx0, sd0)
            s_start(rows0, didx0, ss0)
            g_wait(k + 1, rows1, sg1)
            d_wait(k + 1, didx1, sd1)
            s_start(rows1, didx1, ss1)
            s_wait(rows0, didx0, ss0)
            g_start(k + 2, rows0, sg0)
            d_start(k + 2, didx0, sd0)
            s_wait(rows1, didx1, ss1)
            g_start(k + 3, rows1, sg1)
            d_start(k + 3, didx1, sd1)

        k = EPT_CHUNKS - 2
        g_wait(k, rows0, sg0)
        d_wait(k, didx0, sd0)
        s_start(rows0, didx0, ss0)
        g_wait(k + 1, rows1, sg1)
        d_wait(k + 1, didx1, sd1)
        s_start(rows1, didx1, ss1)
        s_wait(rows0, didx0, ss0)
        s_wait(rows1, didx1, ss1)

        plsc.subcore_barrier()
        pltpu.sync_copy(agg_sh.at[pl.ds(s * RPT, RPT)],
                        out_hbm.at[c, pl.ds(s * RPT, RPT)])

    return scatter_kernel(y, src_idx3, dst_idx3, zeros_rows)


def _matmul_body(x_ref, w_ref, deg_ref, y_ref):
    dsum = deg_ref[:, 0:1] + deg_ref[:, 1:2] + 1.0
    xw = jnp.dot(x_ref[...], w_ref[...], preferred_element_type=jnp.float32)
    y_ref[...] = xw * lax.rsqrt(dsum)


def _tc_scale_matmul(x_pad, W, deg_t):
    return pl.pallas_call(
        _matmul_body,
        grid=(NPAD // BR,),
        in_specs=[
            pl.BlockSpec((BR, D), lambda i: (i, 0)),
            pl.BlockSpec((D, D), lambda i: (0, 0)),
            pl.BlockSpec((BR, NC), lambda i: (i, 0)),
        ],
        out_specs=pl.BlockSpec((BR, D), lambda i: (i, 0)),
        out_shape=jax.ShapeDtypeStruct((NPAD, D), jnp.float32),
    )(x_pad, W, deg_t)


def _final_body(agg_ref, y_ref, deg_ref, b_ref, g_ref, bt_ref, o_ref):
    dsum = deg_ref[:, 0:1] + deg_ref[:, 1:2] + 1.0
    t = (agg_ref[0] + agg_ref[1] + y_ref[...]) * lax.rsqrt(dsum) + b_ref[...]
    mean = jnp.mean(t, axis=-1, keepdims=True)
    ctr = t - mean
    var = jnp.mean(ctr * ctr, axis=-1, keepdims=True)
    o_ref[...] = ctr * lax.rsqrt(var + 1e-5) * g_ref[...] + bt_ref[...]


def _tc_final(agg_parts, y, deg_t, b2, g2, bt2):
    return pl.pallas_call(
        _final_body,
        grid=(NPAD // BR,),
        in_specs=[
            pl.BlockSpec((NC, BR, D), lambda i: (0, i, 0)),
            pl.BlockSpec((BR, D), lambda i: (i, 0)),
            pl.BlockSpec((BR, NC), lambda i: (i, 0)),
            pl.BlockSpec((1, D), lambda i: (0, 0)),
            pl.BlockSpec((1, D), lambda i: (0, 0)),
            pl.BlockSpec((1, D), lambda i: (0, 0)),
        ],
        out_specs=pl.BlockSpec((BR, D), lambda i: (i, 0)),
        out_shape=jax.ShapeDtypeStruct((NPAD, D), jnp.float32),
    )(agg_parts, y, deg_t, b2, g2, bt2)


def kernel(x, edge_index, W, b, gamma, beta):
    src = edge_index[0].astype(jnp.int32)
    dst = edge_index[1].astype(jnp.int32)
    pad_e = EPAD - N_EDGES
    fill = jnp.arange(pad_e, dtype=jnp.int32)
    src_pad = jnp.concatenate([src, fill % N_NODES])
    dst_pad = jnp.concatenate([dst, N_NODES + fill % (NPAD - N_NODES)])
    src2 = src_pad.reshape(NW, EPT)
    dst3 = dst_pad.reshape(NW, EPT_CHUNKS, CHUNK)

    zeros_col = jnp.zeros((RPT,), jnp.float32)
    ones_chunk = jnp.ones((CHUNK,), jnp.float32)
    zeros_rows = jnp.zeros((RPT, D), jnp.float32)

    deg_parts = _sc_degree(dst3, zeros_col, ones_chunk)
    deg_t = deg_parts.T

    x_pad = jnp.concatenate([x, jnp.zeros((NPAD - N_NODES, D), x.dtype)])
    y = _tc_scale_matmul(x_pad, W, deg_t)
    agg_parts = _sc_scatter(y, src2, dst3, zeros_rows)
    out = _tc_final(agg_parts, y, deg_t,
                    b.reshape(1, D), gamma.reshape(1, D), beta.reshape(1, D))
    return out[:N_NODES]


# speedup vs baseline: 36.0162x; 1.0356x over previous
import functools

import jax
import jax.numpy as jnp
from jax import lax
from jax.experimental import pallas as pl
from jax.experimental.pallas import tpu as pltpu
from jax.experimental.pallas import tpu_sc as plsc

N_NODES = 10000
N_EDGES = 320000
D = 128

NC = 2
NS = 16
NW = NC * NS

NPAD = 10240
RPT = NPAD // NS
CHUNK = 128
EPT_CHUNKS = 80
EPT = CHUNK * EPT_CHUNKS
EPAD = EPT * NW
BR = 1000


def _sc_degree(dst_idx3, zeros_col, ones_chunk):
    mesh = plsc.VectorSubcoreMesh(core_axis_name="c", subcore_axis_name="s")

    @functools.partial(
        pl.kernel,
        out_type=jax.ShapeDtypeStruct((NC, NPAD), jnp.float32),
        mesh=mesh,
        scratch_types=[
            pltpu.VMEM((EPT_CHUNKS, CHUNK), jnp.int32),
            pltpu.VMEM((CHUNK,), jnp.float32),
            pltpu.SemaphoreType.DMA,
            pltpu.VMEM_SHARED((NPAD,), jnp.float32),
        ],
    )
    def deg_kernel(dst_hbm, z_hbm, one_hbm, out_hbm, didx_all, ones_v, sem, deg_sh):
        c = lax.axis_index("c")
        s = lax.axis_index("s")
        wid = c * NS + s
        pltpu.sync_copy(dst_hbm.at[wid], didx_all)
        pltpu.sync_copy(z_hbm, deg_sh.at[pl.ds(s * RPT, RPT)])
        pltpu.sync_copy(one_hbm, ones_v)
        plsc.subcore_barrier()

        @pl.loop(0, EPT_CHUNKS)
        def _(i):
            pltpu.async_copy(ones_v, deg_sh.at[didx_all.at[i]], sem, add=True)

        @pl.loop(0, EPT_CHUNKS)
        def _(i):
            pltpu.make_async_copy(ones_v, deg_sh.at[didx_all.at[i]], sem).wait()

        plsc.subcore_barrier()
        pltpu.sync_copy(deg_sh.at[pl.ds(s * RPT, RPT)],
                        out_hbm.at[c, pl.ds(s * RPT, RPT)])

    return deg_kernel(dst_idx3, zeros_col, ones_chunk)


def _sc_scatter(y, src_idx3, dst_idx3, zeros_rows):
    mesh = plsc.VectorSubcoreMesh(core_axis_name="c", subcore_axis_name="s")

    @functools.partial(
        pl.kernel,
        out_type=jax.ShapeDtypeStruct((NC, NPAD, D), jnp.float32),
        mesh=mesh,
        scratch_types=[
            pltpu.VMEM((EPT,), jnp.int32),
            pltpu.VMEM((CHUNK,), jnp.int32),
            pltpu.VMEM((CHUNK,), jnp.int32),
            pltpu.VMEM((CHUNK, D), jnp.float32),
            pltpu.VMEM((CHUNK, D), jnp.float32),
            pltpu.SemaphoreType.DMA,
            pltpu.SemaphoreType.DMA,
            pltpu.SemaphoreType.DMA,
            pltpu.SemaphoreType.DMA,
            pltpu.SemaphoreType.DMA,
            pltpu.SemaphoreType.DMA,
            pltpu.VMEM_SHARED((NPAD, D), jnp.float32),
        ],
    )
    def scatter_kernel(y_hbm, src_hbm, dst_hbm, z_hbm, out_hbm,
                       sidx_all, didx0, didx1, rows0, rows1,
                       sg0, sg1, ss0, ss1, sd0, sd1, agg_sh):
        c = lax.axis_index("c")
        s = lax.axis_index("s")
        wid = c * NS + s
        pltpu.sync_copy(src_hbm.at[wid], sidx_all)
        pltpu.sync_copy(z_hbm, agg_sh.at[pl.ds(s * RPT, RPT)])
        plsc.subcore_barrier()

        def g_start(k, buf, sem):
            pltpu.async_copy(y_hbm.at[sidx_all.at[pl.ds(k * CHUNK, CHUNK)]],
                             buf, sem)

        def g_wait(k, buf, sem):
            pltpu.make_async_copy(y_hbm.at[sidx_all.at[pl.ds(k * CHUNK, CHUNK)]],
                                  buf, sem).wait()

        def d_start(k, dbuf, sem):
            pltpu.async_copy(dst_hbm.at[wid, k], dbuf, sem)

        def d_wait(k, dbuf, sem):
            pltpu.make_async_copy(dst_hbm.at[wid, k], dbuf, sem).wait()

        def s_start(buf, dbuf, sem):
            pltpu.async_copy(buf, agg_sh.at[dbuf], sem, add=True)

        def s_wait(buf, dbuf, sem):
            pltpu.make_async_copy(buf, agg_sh.at[dbuf], sem).wait()

        g_start(0, rows0, sg0)
        d_start(0, didx0, sd0)
        g_start(1, rows1, sg1)
        d_start(1, didx1, sd1)

        @pl.loop(0, EPT_CHUNKS // 2 - 1)
        def _(i):
            k = i * 2
            g_wait(k, rows0, sg0)
            d_wait(k, didx0, sd0)
            s_start(rows0, didx0, ss0)
            g_wait(k + 1, rows1, sg1)
            d_wait(k + 1, didx1, sd1)
            s_start(rows1, didx1, ss1)
            s_wait(rows0, didx0, ss0)
            g_start(k + 2, rows0, sg0)
            d_start(k + 2, didx0, sd0)
            s_wait(rows1, didx1, ss1)
            g_start(k + 3, rows1, sg1)
            d_start(k + 3, didx1, sd1)

        k = EPT_CHUNKS - 2
        g_wait(k, rows0, sg0)
        d_wait(k, didx0, sd0)
        s_start(rows0, didx0, ss0)
        g_wait(k + 1, rows1, sg1)
        d_wait(k + 1, didx1, sd1)
        s_start(rows1, didx1, ss1)
        s_wait(rows0, didx0, ss0)
        s_wait(rows1, didx1, ss1)

        plsc.subcore_barrier()
        pltpu.sync_copy(agg_sh.at[pl.ds(s * RPT, RPT)],
                        out_hbm.at[c, pl.ds(s * RPT, RPT)])

    return scatter_kernel(y, src_idx3, dst_idx3, zeros_rows)


def _matmul_body(x_ref, w_ref, deg_ref, y_ref):
    dsum = deg_ref[:, 0:1] + deg_ref[:, 1:2] + 1.0
    xw = jnp.dot(x_ref[...], w_ref[...], preferred_element_type=jnp.float32)
    y_ref[...] = xw * lax.rsqrt(dsum)


def _tc_scale_matmul(x, W, deg_t):
    return pl.pallas_call(
        _matmul_body,
        grid=(N_NODES // BR,),
        in_specs=[
            pl.BlockSpec((BR, D), lambda i: (i, 0)),
            pl.BlockSpec((D, D), lambda i: (0, 0)),
            pl.BlockSpec((BR, NC), lambda i: (i, 0)),
        ],
        out_specs=pl.BlockSpec((BR, D), lambda i: (i, 0)),
        out_shape=jax.ShapeDtypeStruct((N_NODES, D), jnp.float32),
    )(x, W, deg_t)


def _final_body(agg_ref, y_ref, deg_ref, b_ref, g_ref, bt_ref, o_ref):
    dsum = deg_ref[:, 0:1] + deg_ref[:, 1:2] + 1.0
    t = (agg_ref[0] + agg_ref[1] + y_ref[...]) * lax.rsqrt(dsum) + b_ref[...]
    mean = jnp.mean(t, axis=-1, keepdims=True)
    ctr = t - mean
    var = jnp.mean(ctr * ctr, axis=-1, keepdims=True)
    o_ref[...] = ctr * lax.rsqrt(var + 1e-5) * g_ref[...] + bt_ref[...]


def _tc_final(agg_parts, y, deg_t, b2, g2, bt2):
    return pl.pallas_call(
        _final_body,
        grid=(N_NODES // BR,),
        in_specs=[
            pl.BlockSpec((NC, BR, D), lambda i: (0, i, 0)),
            pl.BlockSpec((BR, D), lambda i: (i, 0)),
            pl.BlockSpec((BR, NC), lambda i: (i, 0)),
            pl.BlockSpec((1, D), lambda i: (0, 0)),
            pl.BlockSpec((1, D), lambda i: (0, 0)),
            pl.BlockSpec((1, D), lambda i: (0, 0)),
        ],
        out_specs=pl.BlockSpec((BR, D), lambda i: (i, 0)),
        out_shape=jax.ShapeDtypeStruct((N_NODES, D), jnp.float32),
    )(agg_parts, y, deg_t, b2, g2, bt2)


def kernel(x, edge_index, W, b, gamma, beta):
    src = edge_index[0].astype(jnp.int32)
    dst = edge_index[1].astype(jnp.int32)
    pad_e = EPAD - N_EDGES
    fill = jnp.arange(pad_e, dtype=jnp.int32)
    src_pad = jnp.concatenate([src, fill % N_NODES])
    dst_pad = jnp.concatenate([dst, N_NODES + fill % (NPAD - N_NODES)])
    src2 = src_pad.reshape(NW, EPT)
    dst3 = dst_pad.reshape(NW, EPT_CHUNKS, CHUNK)

    zeros_col = jnp.zeros((RPT,), jnp.float32)
    ones_chunk = jnp.ones((CHUNK,), jnp.float32)
    zeros_rows = jnp.zeros((RPT, D), jnp.float32)

    deg_parts = _sc_degree(dst3, zeros_col, ones_chunk)
    deg_t = deg_parts.T

    y = _tc_scale_matmul(x, W, deg_t)
    agg_parts = _sc_scatter(y, src2, dst3, zeros_rows)
    return _tc_final(agg_parts, y, deg_t,
                     b.reshape(1, D), gamma.reshape(1, D), beta.reshape(1, D))


# speedup vs baseline: 41.7480x; 1.1591x over previous
import functools

import jax
import jax.numpy as jnp
from jax import lax
from jax.experimental import pallas as pl
from jax.experimental.pallas import tpu as pltpu
from jax.experimental.pallas import tpu_sc as plsc

N_NODES = 10000
N_EDGES = 320000
D = 128

NC = 2
NS = 16
NW = NC * NS

NPAD = 10240
RPT = NPAD // NS
CHUNK = 64
EPT_CHUNKS = 160
EPT = CHUNK * EPT_CHUNKS
EPAD = EPT * NW
BR = 1000
NBUF = 4


def _sc_degree(dst_idx3, zeros_col, ones_chunk):
    mesh = plsc.VectorSubcoreMesh(core_axis_name="c", subcore_axis_name="s")

    @functools.partial(
        pl.kernel,
        out_type=jax.ShapeDtypeStruct((NC, NPAD), jnp.float32),
        mesh=mesh,
        scratch_types=[
            pltpu.VMEM((EPT_CHUNKS, CHUNK), jnp.int32),
            pltpu.VMEM((CHUNK,), jnp.float32),
            pltpu.SemaphoreType.DMA,
            pltpu.VMEM_SHARED((NPAD,), jnp.float32),
        ],
    )
    def deg_kernel(dst_hbm, z_hbm, one_hbm, out_hbm, didx_all, ones_v, sem, deg_sh):
        c = lax.axis_index("c")
        s = lax.axis_index("s")
        wid = c * NS + s
        pltpu.sync_copy(dst_hbm.at[wid], didx_all)
        pltpu.sync_copy(z_hbm, deg_sh.at[pl.ds(s * RPT, RPT)])
        pltpu.sync_copy(one_hbm, ones_v)
        plsc.subcore_barrier()

        @pl.loop(0, EPT_CHUNKS)
        def _(i):
            pltpu.async_copy(ones_v, deg_sh.at[didx_all.at[i]], sem, add=True)

        @pl.loop(0, EPT_CHUNKS)
        def _(i):
            pltpu.make_async_copy(ones_v, deg_sh.at[didx_all.at[i]], sem).wait()

        plsc.subcore_barrier()
        pltpu.sync_copy(deg_sh.at[pl.ds(s * RPT, RPT)],
                        out_hbm.at[c, pl.ds(s * RPT, RPT)])

    return deg_kernel(dst_idx3, zeros_col, ones_chunk)


def _sc_scatter(yq, src_idx2, dst_idx3, zeros_rows):
    mesh = plsc.VectorSubcoreMesh(core_axis_name="c", subcore_axis_name="s")

    @functools.partial(
        pl.kernel,
        out_type=jax.ShapeDtypeStruct((NC, NPAD, D), jnp.float32),
        mesh=mesh,
        scratch_types=(
            [pltpu.VMEM((EPT,), jnp.int32)]
            + [pltpu.VMEM((CHUNK,), jnp.int32) for _ in range(NBUF)]
            + [pltpu.VMEM((CHUNK, D), jnp.float32) for _ in range(NBUF)]
            + [pltpu.SemaphoreType.DMA for _ in range(3 * NBUF)]
            + [pltpu.VMEM_SHARED((NPAD, D), jnp.float32)]
        ),
    )
    def scatter_kernel(y_hbm, src_hbm, dst_hbm, z_hbm, out_hbm, sidx_all, *rest):
        didx = rest[:NBUF]
        rows = rest[NBUF:2 * NBUF]
        sg = rest[2 * NBUF:3 * NBUF]
        ss = rest[3 * NBUF:4 * NBUF]
        sd = rest[4 * NBUF:5 * NBUF]
        agg_sh = rest[5 * NBUF]
        c = lax.axis_index("c")
        s = lax.axis_index("s")
        wid = c * NS + s
        pltpu.sync_copy(src_hbm.at[wid], sidx_all)
        pltpu.sync_copy(z_hbm, agg_sh.at[pl.ds(s * RPT, RPT)])
        plsc.subcore_barrier()

        def g_start(k, b):
            pltpu.async_copy(y_hbm.at[sidx_all.at[pl.ds(k * CHUNK, CHUNK)]],
                             rows[b], sg[b])

        def g_wait(k, b):
            pltpu.make_async_copy(y_hbm.at[sidx_all.at[pl.ds(k * CHUNK, CHUNK)]],
                                  rows[b], sg[b]).wait()

        def d_start(k, b):
            pltpu.async_copy(dst_hbm.at[wid, k], didx[b], sd[b])

        def d_wait(k, b):
            pltpu.make_async_copy(dst_hbm.at[wid, k], didx[b], sd[b]).wait()

        def s_start(b):
            pltpu.async_copy(rows[b], agg_sh.at[didx[b]], ss[b], add=True)

        def s_wait(b):
            pltpu.make_async_copy(rows[b], agg_sh.at[didx[b]], ss[b]).wait()

        for b in range(NBUF):
            g_start(b, b)
            d_start(b, b)

        @pl.loop(0, EPT_CHUNKS // NBUF - 1)
        def _(i):
            k0 = i * NBUF
            for b in range(NBUF):
                g_wait(k0 + b, b)
                d_wait(k0 + b, b)
                s_start(b)
            for b in range(NBUF):
                s_wait(b)
                g_start(k0 + NBUF + b, b)
                d_start(k0 + NBUF + b, b)

        k0 = EPT_CHUNKS - NBUF
        for b in range(NBUF):
            g_wait(k0 + b, b)
            d_wait(k0 + b, b)
            s_start(b)
        for b in range(NBUF):
            s_wait(b)

        plsc.subcore_barrier()
        pltpu.sync_copy(agg_sh.at[pl.ds(s * RPT, RPT)],
                        out_hbm.at[c, pl.ds(s * RPT, RPT)])

    return scatter_kernel(yq, src_idx2, dst_idx3, zeros_rows)


def _matmul_body(x_ref, w_ref, deg_ref, y_ref):
    dsum = deg_ref[:, 0:1] + deg_ref[:, 1:2] + 1.0
    xw = jnp.dot(x_ref[...], w_ref[...], preferred_element_type=jnp.float32)
    y_ref[...] = xw * lax.rsqrt(dsum)


def _tc_scale_matmul(x, W, deg_t):
    return pl.pallas_call(
        _matmul_body,
        grid=(N_NODES // BR,),
        in_specs=[
            pl.BlockSpec((BR, D), lambda i: (i, 0)),
            pl.BlockSpec((D, D), lambda i: (0, 0)),
            pl.BlockSpec((BR, NC), lambda i: (i, 0)),
        ],
        out_specs=pl.BlockSpec((BR, D), lambda i: (i, 0)),
        out_shape=jax.ShapeDtypeStruct((N_NODES, D), jnp.float32),
    )(x, W, deg_t)


def _final_body(agg_ref, y_ref, deg_ref, b_ref, g_ref, bt_ref, o_ref):
    dsum = deg_ref[:, 0:1] + deg_ref[:, 1:2] + 1.0
    t = (agg_ref[0] + agg_ref[1] + y_ref[...]) * lax.rsqrt(dsum) + b_ref[...]
    mean = jnp.mean(t, axis=-1, keepdims=True)
    ctr = t - mean
    var = jnp.mean(ctr * ctr, axis=-1, keepdims=True)
    o_ref[...] = ctr * lax.rsqrt(var + 1e-5) * g_ref[...] + bt_ref[...]


def _tc_final(agg_parts, y, deg_t, b2, g2, bt2):
    return pl.pallas_call(
        _final_body,
        grid=(N_NODES // BR,),
        in_specs=[
            pl.BlockSpec((NC, BR, D), lambda i: (0, i, 0)),
            pl.BlockSpec((BR, D), lambda i: (i, 0)),
            pl.BlockSpec((BR, NC), lambda i: (i, 0)),
            pl.BlockSpec((1, D), lambda i: (0, 0)),
            pl.BlockSpec((1, D), lambda i: (0, 0)),
            pl.BlockSpec((1, D), lambda i: (0, 0)),
        ],
        out_specs=pl.BlockSpec((BR, D), lambda i: (i, 0)),
        out_shape=jax.ShapeDtypeStruct((N_NODES, D), jnp.float32),
    )(agg_parts, y, deg_t, b2, g2, bt2)


def kernel(x, edge_index, W, b, gamma, beta):
    src = edge_index[0].astype(jnp.int32)
    dst = edge_index[1].astype(jnp.int32)
    pad_e = EPAD - N_EDGES
    fill = jnp.arange(pad_e, dtype=jnp.int32)
    src_pad = jnp.concatenate([src, fill % N_NODES])
    dst_pad = jnp.concatenate([dst, N_NODES + fill % (NPAD - N_NODES)])
    src2 = src_pad.reshape(NW, EPT)
    dst3 = dst_pad.reshape(NW, EPT_CHUNKS, CHUNK)

    zeros_col = jnp.zeros((RPT,), jnp.float32)
    ones_chunk = jnp.ones((CHUNK,), jnp.float32)
    zeros_rows = jnp.zeros((RPT, D), jnp.float32)

    deg_parts = _sc_degree(dst3, zeros_col, ones_chunk)
    deg_t = deg_parts.T

    y = _tc_scale_matmul(x, W, deg_t)
    agg_parts = _sc_scatter(y, src2, dst3, zeros_rows)
    return _tc_final(agg_parts, y, deg_t,
                     b.reshape(1, D), gamma.reshape(1, D), beta.reshape(1, D))


# speedup vs baseline: 43.7213x; 1.0473x over previous
import functools

import jax
import jax.numpy as jnp
from jax import lax
from jax.experimental import pallas as pl
from jax.experimental.pallas import tpu as pltpu
from jax.experimental.pallas import tpu_sc as plsc

N_NODES = 10000
N_EDGES = 320000
D = 128

NC = 2
NS = 16
NW = NC * NS

NPAD = 10240
RPT = NPAD // NS
ROW = 128
ECH = N_EDGES // ROW
CPT = ECH // NW
NTAIL = ECH - CPT * NW
CHUNK = 64
NSUB = CPT * 2
NBUF = 3
BR = 2000


def _sc_degree(eq, zeros_col, ones_row):
    mesh = plsc.VectorSubcoreMesh(core_axis_name="c", subcore_axis_name="s")

    @functools.partial(
        pl.kernel,
        out_type=jax.ShapeDtypeStruct((NC, NPAD), jnp.float32),
        mesh=mesh,
        scratch_types=[
            pltpu.VMEM((CPT, 2, ROW), jnp.int32),
            pltpu.VMEM((2, ROW), jnp.int32),
            pltpu.VMEM((ROW,), jnp.float32),
            pltpu.SemaphoreType.DMA,
            pltpu.VMEM_SHARED((NPAD,), jnp.float32),
        ],
    )
    def deg_kernel(eq_hbm, z_hbm, one_hbm, out_hbm, eqpre, ebuf, ones_v, sem,
                   deg_sh):
        c = lax.axis_index("c")
        s = lax.axis_index("s")
        wid = c * NS + s
        pltpu.sync_copy(eq_hbm.at[pl.ds(wid * CPT, CPT)], eqpre)
        pltpu.sync_copy(z_hbm, deg_sh.at[pl.ds(s * RPT, RPT)])
        pltpu.sync_copy(one_hbm, ones_v)
        plsc.subcore_barrier()

        @pl.loop(0, CPT)
        def _(j):
            pltpu.async_copy(ones_v, deg_sh.at[eqpre.at[j, 1]], sem, add=True)

        @pl.loop(0, CPT)
        def _(j):
            pltpu.make_async_copy(ones_v, deg_sh.at[eqpre.at[j, 1]], sem).wait()

        @pl.when(wid < NTAIL)
        def _():
            pltpu.sync_copy(eq_hbm.at[NW * CPT + wid], ebuf)
            pltpu.sync_copy(ones_v, deg_sh.at[ebuf.at[1]], add=True)

        plsc.subcore_barrier()
        pltpu.sync_copy(deg_sh.at[pl.ds(s * RPT, RPT)],
                        out_hbm.at[c, pl.ds(s * RPT, RPT)])

    return deg_kernel(eq, zeros_col, ones_row)


def _sc_scatter(y, eq, zeros_rows):
    mesh = plsc.VectorSubcoreMesh(core_axis_name="c", subcore_axis_name="s")

    @functools.partial(
        pl.kernel,
        out_type=jax.ShapeDtypeStruct((NC, NPAD, D), jnp.float32),
        mesh=mesh,
        scratch_types=(
            [pltpu.VMEM((CPT, 2, ROW), jnp.int32),
             pltpu.VMEM((2, ROW), jnp.int32)]
            + [pltpu.VMEM((CHUNK, D), jnp.float32) for _ in range(NBUF)]
            + [pltpu.SemaphoreType.DMA for _ in range(2 * NBUF)]
            + [pltpu.VMEM_SHARED((NPAD, D), jnp.float32)]
        ),
    )
    def scatter_kernel(y_hbm, eq_hbm, z_hbm, out_hbm, eqpre, ebuf, *rest):
        rows = rest[:NBUF]
        sg = rest[NBUF:2 * NBUF]
        ss = rest[2 * NBUF:3 * NBUF]
        agg_sh = rest[3 * NBUF]
        c = lax.axis_index("c")
        s = lax.axis_index("s")
        wid = c * NS + s
        pltpu.sync_copy(eq_hbm.at[pl.ds(wid * CPT, CPT)], eqpre)
        pltpu.sync_copy(z_hbm, agg_sh.at[pl.ds(s * RPT, RPT)])
        plsc.subcore_barrier()

        def sidx(k):
            return eqpre.at[k // 2, 0, pl.ds((k % 2) * CHUNK, CHUNK)]

        def didx(k):
            return eqpre.at[k // 2, 1, pl.ds((k % 2) * CHUNK, CHUNK)]

        def g_start(k, b):
            pltpu.async_copy(y_hbm.at[sidx(k)], rows[b], sg[b])

        def g_wait(k, b):
            pltpu.make_async_copy(y_hbm.at[sidx(k)], rows[b], sg[b]).wait()

        def s_start(k, b):
            pltpu.async_copy(rows[b], agg_sh.at[didx(k)], ss[b], add=True)

        def s_wait(k, b):
            pltpu.make_async_copy(rows[b], agg_sh.at[didx(k)], ss[b]).wait()

        for b in range(NBUF):
            g_start(b, b)

        @pl.loop(0, NSUB // NBUF - 1)
        def _(i):
            k0 = i * NBUF
            for b in range(NBUF):
                g_wait(k0 + b, b)
                s_start(k0 + b, b)
            for b in range(NBUF):
                s_wait(k0 + b, b)
                g_start(k0 + NBUF + b, b)

        k0 = NSUB - NBUF
        for b in range(NBUF):
            g_wait(k0 + b, b)
            s_start(k0 + b, b)
        for b in range(NBUF):
            s_wait(k0 + b, b)

        @pl.when(wid < NTAIL)
        def _():
            pltpu.sync_copy(eq_hbm.at[NW * CPT + wid], ebuf)
            for h in range(2):
                pltpu.sync_copy(y_hbm.at[ebuf.at[0, pl.ds(h * CHUNK, CHUNK)]],
                                rows[0])
                pltpu.sync_copy(rows[0],
                                agg_sh.at[ebuf.at[1, pl.ds(h * CHUNK, CHUNK)]],
                                add=True)

        plsc.subcore_barrier()
        pltpu.sync_copy(agg_sh.at[pl.ds(s * RPT, RPT)],
                        out_hbm.at[c, pl.ds(s * RPT, RPT)])

    return scatter_kernel(y, eq, zeros_rows)


def _matmul_body(x_ref, w_ref, deg_ref, y_ref):
    dsum = deg_ref[:, 0:1] + deg_ref[:, 1:2] + 1.0
    xw = jnp.dot(x_ref[...], w_ref[...], preferred_element_type=jnp.float32)
    y_ref[...] = xw * lax.rsqrt(dsum)


def _tc_scale_matmul(x, W, deg_t):
    return pl.pallas_call(
        _matmul_body,
        grid=(N_NODES // BR,),
        in_specs=[
            pl.BlockSpec((BR, D), lambda i: (i, 0)),
            pl.BlockSpec((D, D), lambda i: (0, 0)),
            pl.BlockSpec((BR, NC), lambda i: (i, 0)),
        ],
        out_specs=pl.BlockSpec((BR, D), lambda i: (i, 0)),
        out_shape=jax.ShapeDtypeStruct((N_NODES, D), jnp.float32),
    )(x, W, deg_t)


def _final_body(agg_ref, y_ref, deg_ref, b_ref, g_ref, bt_ref, o_ref):
    dsum = deg_ref[:, 0:1] + deg_ref[:, 1:2] + 1.0
    t = (agg_ref[0] + agg_ref[1] + y_ref[...]) * lax.rsqrt(dsum) + b_ref[...]
    mean = jnp.mean(t, axis=-1, keepdims=True)
    ctr = t - mean
    var = jnp.mean(ctr * ctr, axis=-1, keepdims=True)
    o_ref[...] = ctr * lax.rsqrt(var + 1e-5) * g_ref[...] + bt_ref[...]


def _tc_final(agg_parts, y, deg_t, b2, g2, bt2):
    return pl.pallas_call(
        _final_body,
        grid=(N_NODES // BR,),
        in_specs=[
            pl.BlockSpec((NC, BR, D), lambda i: (0, i, 0)),
            pl.BlockSpec((BR, D), lambda i: (i, 0)),
            pl.BlockSpec((BR, NC), lambda i: (i, 0)),
            pl.BlockSpec((1, D), lambda i: (0, 0)),
            pl.BlockSpec((1, D), lambda i: (0, 0)),
            pl.BlockSpec((1, D), lambda i: (0, 0)),
        ],
        out_specs=pl.BlockSpec((BR, D), lambda i: (i, 0)),
        out_shape=jax.ShapeDtypeStruct((N_NODES, D), jnp.float32),
    )(agg_parts, y, deg_t, b2, g2, bt2)


def kernel(x, edge_index, W, b, gamma, beta):
    eq = edge_index.astype(jnp.int32).reshape(2, ECH, ROW).transpose(1, 0, 2)

    zeros_col = jnp.zeros((RPT,), jnp.float32)
    ones_row = jnp.ones((ROW,), jnp.float32)
    zeros_rows = jnp.zeros((RPT, D), jnp.float32)

    deg_t = _sc_degree(eq, zeros_col, ones_row).T
    y = _tc_scale_matmul(x, W, deg_t)
    agg_parts = _sc_scatter(y, eq, zeros_rows)
    return _tc_final(agg_parts, y, deg_t,
                     b.reshape(1, D), gamma.reshape(1, D), beta.reshape(1, D))


# speedup vs baseline: 45.7944x; 1.0474x over previous
import functools

import numpy as np

import jax
import jax.numpy as jnp
from jax import lax
from jax.experimental import pallas as pl
from jax.experimental.pallas import tpu as pltpu
from jax.experimental.pallas import tpu_sc as plsc

N_NODES = 10000
N_EDGES = 320000
D = 128

NC = 2
NS = 16
NW = NC * NS

NPAD = 10240
RPT = NPAD // NS
ROW = 128
ECH = N_EDGES // ROW
CPT = ECH // NW
NTAIL = ECH - CPT * NW
CHUNK = 64
NSUB = CPT * 2
NBUF = 4
BR = 2000

_Z_COL = np.zeros((RPT,), np.float32)
_ONES_ROW = np.ones((ROW,), np.float32)
_Z_ROWS = np.zeros((RPT, D), np.float32)


def _sc_degree(eq, zeros_col, ones_row):
    mesh = plsc.VectorSubcoreMesh(core_axis_name="c", subcore_axis_name="s")

    @functools.partial(
        pl.kernel,
        out_type=jax.ShapeDtypeStruct((NC, NPAD), jnp.float32),
        mesh=mesh,
        scratch_types=[
            pltpu.VMEM((CPT, 2, ROW), jnp.int32),
            pltpu.VMEM((2, ROW), jnp.int32),
            pltpu.VMEM((ROW,), jnp.float32),
            pltpu.SemaphoreType.DMA,
            pltpu.VMEM_SHARED((NPAD,), jnp.float32),
        ],
    )
    def deg_kernel(eq_hbm, z_hbm, one_hbm, out_hbm, eqpre, ebuf, ones_v, sem,
                   deg_sh):
        c = lax.axis_index("c")
        s = lax.axis_index("s")
        wid = c * NS + s
        pltpu.sync_copy(eq_hbm.at[pl.ds(wid * CPT, CPT)], eqpre)
        pltpu.sync_copy(z_hbm, deg_sh.at[pl.ds(s * RPT, RPT)])
        pltpu.sync_copy(one_hbm, ones_v)
        plsc.subcore_barrier()

        @pl.loop(0, CPT)
        def _(j):
            pltpu.async_copy(ones_v, deg_sh.at[eqpre.at[j, 1]], sem, add=True)

        @pl.loop(0, CPT)
        def _(j):
            pltpu.make_async_copy(ones_v, deg_sh.at[eqpre.at[j, 1]], sem).wait()

        @pl.when(wid < NTAIL)
        def _():
            pltpu.sync_copy(eq_hbm.at[NW * CPT + wid], ebuf)
            pltpu.sync_copy(ones_v, deg_sh.at[ebuf.at[1]], add=True)

        plsc.subcore_barrier()
        pltpu.sync_copy(deg_sh.at[pl.ds(s * RPT, RPT)],
                        out_hbm.at[c, pl.ds(s * RPT, RPT)])

    return deg_kernel(eq, zeros_col, ones_row)


def _sc_scatter(y, eq, zeros_rows):
    mesh = plsc.VectorSubcoreMesh(core_axis_name="c", subcore_axis_name="s")

    @functools.partial(
        pl.kernel,
        out_type=jax.ShapeDtypeStruct((NC, NPAD, D), jnp.float32),
        mesh=mesh,
        scratch_types=(
            [pltpu.VMEM((CPT, ROW), jnp.int32),
             pltpu.VMEM((2, ROW), jnp.int32)]
            + [pltpu.VMEM((2, ROW), jnp.int32) for _ in range(NBUF)]
            + [pltpu.VMEM((CHUNK, D), jnp.float32) for _ in range(NBUF)]
            + [pltpu.SemaphoreType.DMA for _ in range(3 * NBUF)]
            + [pltpu.VMEM_SHARED((NPAD, D), jnp.float32)]
        ),
    )
    def scatter_kernel(y_hbm, eq_hbm, z_hbm, out_hbm, spre, ebuf, *rest):
        dbuf = rest[:NBUF]
        rows = rest[NBUF:2 * NBUF]
        sg = rest[2 * NBUF:3 * NBUF]
        ss = rest[3 * NBUF:4 * NBUF]
        sd = rest[4 * NBUF:5 * NBUF]
        agg_sh = rest[5 * NBUF]
        c = lax.axis_index("c")
        s = lax.axis_index("s")
        wid = c * NS + s
        pltpu.sync_copy(eq_hbm.at[pl.ds(wid * CPT, CPT), 0], spre)
        pltpu.sync_copy(z_hbm, agg_sh.at[pl.ds(s * RPT, RPT)])
        plsc.subcore_barrier()

        def sidx(k):
            return spre.at[k // 2, pl.ds((k % 2) * CHUNK, CHUNK)]

        def didx(k, b):
            return dbuf[b].at[1, pl.ds((k % 2) * CHUNK, CHUNK)]

        def g_start(k, b):
            pltpu.async_copy(y_hbm.at[sidx(k)], rows[b], sg[b])

        def g_wait(k, b):
            pltpu.make_async_copy(y_hbm.at[sidx(k)], rows[b], sg[b]).wait()

        def d_start(k, b):
            pltpu.async_copy(eq_hbm.at[wid * CPT + k // 2], dbuf[b], sd[b])

        def d_wait(k, b):
            pltpu.make_async_copy(eq_hbm.at[wid * CPT + k // 2], dbuf[b],
                                  sd[b]).wait()

        def s_start(k, b):
            pltpu.async_copy(rows[b], agg_sh.at[didx(k, b)], ss[b], add=True)

        def s_wait(k, b):
            pltpu.make_async_copy(rows[b], agg_sh.at[didx(k, b)], ss[b]).wait()

        for b in range(NBUF):
            g_start(b, b)
            d_start(b, b)

        @pl.loop(0, NSUB // NBUF - 1)
        def _(i):
            k0 = i * NBUF
            for b in range(NBUF):
                g_wait(k0 + b, b)
                d_wait(k0 + b, b)
                s_start(k0 + b, b)
            for b in range(NBUF):
                s_wait(k0 + b, b)
                g_start(k0 + NBUF + b, b)
                d_start(k0 + NBUF + b, b)

        k0 = NSUB - NBUF
        for b in range(NBUF):
            g_wait(k0 + b, b)
            d_wait(k0 + b, b)
            s_start(k0 + b, b)
        for b in range(NBUF):
            s_wait(k0 + b, b)

        @pl.when(wid < NTAIL)
        def _():
            pltpu.sync_copy(eq_hbm.at[NW * CPT + wid], ebuf)
            for h in range(2):
                pltpu.sync_copy(y_hbm.at[ebuf.at[0, pl.ds(h * CHUNK, CHUNK)]],
                                rows[0])
                pltpu.sync_copy(rows[0],
                                agg_sh.at[ebuf.at[1, pl.ds(h * CHUNK, CHUNK)]],
                                add=True)

        plsc.subcore_barrier()
        pltpu.sync_copy(agg_sh.at[pl.ds(s * RPT, RPT)],
                        out_hbm.at[c, pl.ds(s * RPT, RPT)])

    return scatter_kernel(y, eq, zeros_rows)


def _matmul_body(x_ref, w_ref, deg_ref, y_ref):
    dsum = deg_ref[:, 0:1] + deg_ref[:, 1:2] + 1.0
    xw = jnp.dot(x_ref[...], w_ref[...], preferred_element_type=jnp.float32)
    y_ref[...] = xw * lax.rsqrt(dsum)


def _tc_scale_matmul(x, W, deg_t):
    return pl.pallas_call(
        _matmul_body,
        grid=(N_NODES // BR,),
        in_specs=[
            pl.BlockSpec((BR, D), lambda i: (i, 0)),
            pl.BlockSpec((D, D), lambda i: (0, 0)),
            pl.BlockSpec((BR, NC), lambda i: (i, 0)),
        ],
        out_specs=pl.BlockSpec((BR, D), lambda i: (i, 0)),
        out_shape=jax.ShapeDtypeStruct((N_NODES, D), jnp.float32),
    )(x, W, deg_t)


def _final_body(agg_ref, y_ref, deg_ref, b_ref, g_ref, bt_ref, o_ref):
    dsum = deg_ref[:, 0:1] + deg_ref[:, 1:2] + 1.0
    t = (agg_ref[0] + agg_ref[1] + y_ref[...]) * lax.rsqrt(dsum) + b_ref[...]
    mean = jnp.mean(t, axis=-1, keepdims=True)
    ctr = t - mean
    var = jnp.mean(ctr * ctr, axis=-1, keepdims=True)
    o_ref[...] = ctr * lax.rsqrt(var + 1e-5) * g_ref[...] + bt_ref[...]


def _tc_final(agg_parts, y, deg_t, b2, g2, bt2):
    return pl.pallas_call(
        _final_body,
        grid=(N_NODES // BR,),
        in_specs=[
            pl.BlockSpec((NC, BR, D), lambda i: (0, i, 0)),
            pl.BlockSpec((BR, D), lambda i: (i, 0)),
            pl.BlockSpec((BR, NC), lambda i: (i, 0)),
            pl.BlockSpec((1, D), lambda i: (0, 0)),
            pl.BlockSpec((1, D), lambda i: (0, 0)),
            pl.BlockSpec((1, D), lambda i: (0, 0)),
        ],
        out_specs=pl.BlockSpec((BR, D), lambda i: (i, 0)),
        out_shape=jax.ShapeDtypeStruct((N_NODES, D), jnp.float32),
    )(agg_parts, y, deg_t, b2, g2, bt2)


def kernel(x, edge_index, W, b, gamma, beta):
    eq = edge_index.astype(jnp.int32).reshape(2, ECH, ROW).transpose(1, 0, 2)

    deg_t = _sc_degree(eq, _Z_COL, _ONES_ROW).T
    y = _tc_scale_matmul(x, W, deg_t)
    agg_parts = _sc_scatter(y, eq, _Z_ROWS)
    return _tc_final(agg_parts, y, deg_t,
                     b.reshape(1, D), gamma.reshape(1, D), beta.reshape(1, D))
